# baseline (speedup 1.0000x reference)
import functools

import jax
import jax.numpy as jnp
from jax import lax
from jax.experimental import pallas as pl
from jax.experimental.pallas import tpu as pltpu
from jax.experimental.pallas import tpu_sc as plsc

N = 16384
T = 16
L = 16
NV = N // L
R = 256
HB = R * L
EPS = 1e-12


def _sc_zvalues(predsT, targetsT):
    mesh = plsc.VectorSubcoreMesh(core_axis_name="c", subcore_axis_name="s")

    @functools.partial(
        pl.kernel,
        out_type=jax.ShapeDtypeStruct((T, N), jnp.float32),
        mesh=mesh,
        compiler_params=pltpu.CompilerParams(needs_layout_passes=False),
        scratch_types=[
            pltpu.VMEM((N,), jnp.float32),
            pltpu.VMEM((N,), jnp.float32),
            pltpu.VMEM((N,), jnp.int32),
            pltpu.VMEM((N,), jnp.int32),
            pltpu.VMEM((N,), jnp.float32),
            pltpu.VMEM((HB,), jnp.int32),
            pltpu.VMEM((N,), jnp.float32),
        ],
    )
    def k(predsT_hbm, targetsT_hbm, z_hbm, tgt_v, val_a, key_a, key_b, val_b,
          hist, out_v):
        c = lax.axis_index("c")
        s = lax.axis_index("s")

        @pl.when(s < 8)
        def _():
            task = s * 2 + c
            pltpu.sync_copy(targetsT_hbm.at[task], tgt_v)
            pltpu.sync_copy(predsT_hbm.at[task], val_a)

            ln = lax.iota(jnp.int32, L)
            one = jnp.ones((L,), jnp.int32)

            def zh(j, _):
                hist[pl.ds(j * L, L)] = jnp.zeros((L,), jnp.int32)
                return 0

            lax.fori_loop(0, HB // L, zh, 0, unroll=4)

            def kb(i, mx):
                tv = tgt_v[pl.ds(i * L, L)]
                u = plsc.bitcast(tv, jnp.uint32)
                neg = (u >> 31) != 0
                key = jnp.where(neg, u, u ^ jnp.uint32(0x7FFFFFFF))
                key_a[pl.ds(i * L, L)] = plsc.bitcast(key, jnp.int32)
                d = (key & jnp.uint32(0xFF)).astype(jnp.int32)
                plsc.addupdate_scatter(hist, [(d << 4) + ln], one)
                return jnp.maximum(mx, val_a[pl.ds(i * L, L)])

            mx = lax.fori_loop(0, NV, kb,
                               jnp.full((L,), -jnp.inf, jnp.float32),
                               unroll=4)
            smax = jnp.max(mx)

            def one_pass(shift, src_k, src_v, dst_k, dst_v,
                         skip_hist=False, exp_vals=False, last=False):
                if not skip_hist:
                    def zh2(j, _):
                        hist[pl.ds(j * L, L)] = jnp.zeros((L,), jnp.int32)
                        return 0

                    lax.fori_loop(0, HB // L, zh2, 0, unroll=4)

                    def hb(i, _):
                        ku = plsc.bitcast(src_k[pl.ds(i * L, L)], jnp.uint32)
                        d = ((ku >> jnp.uint32(shift))
                             & jnp.uint32(0xFF)).astype(jnp.int32)
                        plsc.addupdate_scatter(hist, [(d << 4) + ln], one)
                        return 0

                    lax.fori_loop(0, NV, hb, 0, unroll=4)

                def sb(j, carry):
                    v = hist[pl.ds(j * L, L)]
                    cs = plsc.cumsum(v)
                    hist[pl.ds(j * L, L)] = cs - v + carry
                    return carry + jnp.max(cs)

                lax.fori_loop(0, HB // L, sb, jnp.int32(0), unroll=4)

                def pb(i, _):
                    kk = src_k[pl.ds(i * L, L)]
                    vv = src_v[pl.ds(i * L, L)]
                    if exp_vals:
                        vv = jnp.exp(vv - smax)
                    ku = plsc.bitcast(kk, jnp.uint32)
                    d = ((ku >> jnp.uint32(shift))
                         & jnp.uint32(0xFF)).astype(jnp.int32)
                    idx = (d << 4) + ln
                    rank = plsc.load_gather(hist, [idx])
                    pos = ((rank & 1023) << 4) + (rank >> 10)
                    if not last:
                        plsc.store_scatter(dst_k, [pos], kk)
                    plsc.store_scatter(dst_v, [pos], vv)
                    plsc.addupdate_scatter(hist, [idx], one)
                    return 0

                lax.fori_loop(0, NV, pb, 0, unroll=4)

            one_pass(0, key_a, val_a, key_b, val_b, skip_hist=True)
            one_pass(8, key_b, val_b, key_a, val_a)
            one_pass(16, key_a, val_a, key_b, val_b)
            one_pass(24, key_b, val_b, key_a, val_a, exp_vals=True, last=True)

            def suf(j, acc):
                i = NV - 1 - j
                sfx = val_a[pl.ds(i * L, L)] + acc
                out_v[pl.ds(i * L, L)] = sfx
                return sfx

            tot = lax.fori_loop(0, NV, suf, jnp.zeros((L,), jnp.float32),
                                unroll=4)
            car = lax.rev(plsc.cumsum(lax.rev(tot, (0,))), (0,)) - tot

            def addc(i, _):
                out_v[pl.ds(i * L, L)] = out_v[pl.ds(i * L, L)] + car
                return 0

            lax.fori_loop(0, NV, addc, 0, unroll=4)
            pltpu.sync_copy(out_v, z_hbm.at[task])

    return k(predsT, targetsT)


def _tc_finish(z, preds):
    def body(z_ref, p_ref, o_ref):
        lz = jnp.log(z_ref[...] + jnp.float32(EPS))
        o_ref[0, 0] = (jnp.sum(lz) - jnp.sum(p_ref[...])) / jnp.float32(N * T)

    out = pl.pallas_call(
        body,
        out_shape=jax.ShapeDtypeStruct((1, 1), jnp.float32),
        out_specs=pl.BlockSpec(memory_space=pltpu.SMEM),
    )(z, preds)
    return out[0, 0]


def kernel(preds, targets):
    predsT = preds.T.reshape(T, L, NV).transpose(0, 2, 1).reshape(T, N)
    targetsT = targets.T.reshape(T, L, NV).transpose(0, 2, 1).reshape(T, N)
    z = _sc_zvalues(predsT, targetsT)
    return _tc_finish(z, preds)


# speedup vs baseline: 1.5078x; 1.5078x over previous
import functools

import jax
import jax.numpy as jnp
from jax import lax
from jax.experimental import pallas as pl
from jax.experimental.pallas import tpu as pltpu
from jax.experimental.pallas import tpu_sc as plsc

N = 16384
T = 16
L = 16
NV = N // L
R = 2048
EPS = 1e-12


def _sc_zvalues(predsT, targetsT):
    mesh = plsc.VectorSubcoreMesh(core_axis_name="c", subcore_axis_name="s")

    @functools.partial(
        pl.kernel,
        out_type=jax.ShapeDtypeStruct((T, N), jnp.float32),
        mesh=mesh,
        compiler_params=pltpu.CompilerParams(needs_layout_passes=False),
        scratch_types=[
            pltpu.VMEM((N,), jnp.float32),
            pltpu.VMEM((N,), jnp.float32),
            pltpu.VMEM((N,), jnp.int32),
            pltpu.VMEM((N,), jnp.int32),
            pltpu.VMEM((N,), jnp.float32),
            pltpu.VMEM((R,), jnp.int32),
            pltpu.VMEM((N,), jnp.float32),
            pltpu.VMEM((NV + L,), jnp.float32),
        ],
    )
    def k(predsT_hbm, targetsT_hbm, z_hbm, tgt_v, val_a, key_a, key_b, val_b,
          hist, out_v, car_v):
        c = lax.axis_index("c")
        s = lax.axis_index("s")

        @pl.when(s < 8)
        def _():
            task = s * 2 + c
            pltpu.sync_copy(targetsT_hbm.at[task], tgt_v)
            pltpu.sync_copy(predsT_hbm.at[task], val_a)

            occ0, _ = plsc.scan_count(jnp.zeros((L,), jnp.int32))
            base0 = jnp.min(occ0)

            @plsc.parallel_loop(0, R // L)
            def _zh(j):
                hist[pl.ds(j * L, L)] = jnp.zeros((L,), jnp.int32)

            @plsc.parallel_loop(0, NV,
                                carry=jnp.full((L,), -jnp.inf, jnp.float32))
            def mx(i, acc):
                tv = tgt_v[pl.ds(i * L, L)]
                u = plsc.bitcast(tv, jnp.uint32)
                neg = (u >> 31) != 0
                key = jnp.where(neg, u, u ^ jnp.uint32(0x7FFFFFFF))
                key_a[pl.ds(i * L, L)] = plsc.bitcast(key, jnp.int32)
                d = (key & jnp.uint32(0x7FF)).astype(jnp.int32)
                occ, lastm = plsc.scan_count(d)
                plsc.addupdate_scatter(hist, [d], occ - base0 + 1, mask=lastm)
                return jnp.maximum(acc, val_a[pl.ds(i * L, L)])

            smax = jnp.max(mx)

            def one_pass(shift, nbits, src_k, src_v, dst_k, dst_v,
                         skip_hist=False, exp_vals=False, last=False):
                dmask = jnp.uint32((1 << nbits) - 1)

                if not skip_hist:
                    @plsc.parallel_loop(0, R // L)
                    def _zh2(j):
                        hist[pl.ds(j * L, L)] = jnp.zeros((L,), jnp.int32)

                    @plsc.parallel_loop(0, NV)
                    def _hb(i):
                        kk = plsc.bitcast(src_k[pl.ds(i * L, L)], jnp.uint32)
                        d = ((kk >> jnp.uint32(shift)) & dmask).astype(
                            jnp.int32)
                        occ, lastm = plsc.scan_count(d)
                        plsc.addupdate_scatter(hist, [d], occ - base0 + 1,
                                               mask=lastm)

                def sb(j, carry):
                    v = hist[pl.ds(j * L, L)]
                    cs = plsc.cumsum(v)
                    hist[pl.ds(j * L, L)] = cs - v + carry
                    return carry + jnp.sum(v)

                lax.fori_loop(0, R // L, sb, jnp.int32(0), unroll=4)

                def pf(i):
                    kk = src_k[pl.ds(i * L, L)]
                    vv = src_v[pl.ds(i * L, L)]
                    if exp_vals:
                        vv = jnp.exp(vv - smax)
                    ku = plsc.bitcast(kk, jnp.uint32)
                    d = ((ku >> jnp.uint32(shift)) & dmask).astype(jnp.int32)
                    occ, lastm = plsc.scan_count(d)
                    return kk, vv, d, occ, lastm

                def pb(i, st):
                    kk, vv, d, occ, lastm = st
                    base = plsc.load_gather(hist, [d])
                    pos = base + occ - base0
                    if not last:
                        plsc.store_scatter(dst_k, [pos], kk)
                    plsc.store_scatter(dst_v, [pos], vv)
                    plsc.addupdate_scatter(hist, [d], occ - base0 + 1,
                                           mask=lastm)
                    return pf(jnp.minimum(i + 1, NV - 1))

                lax.fori_loop(0, NV, pb, pf(jnp.int32(0)), unroll=4)

            one_pass(0, 11, key_a, val_a, key_b, val_b, skip_hist=True)
            one_pass(11, 11, key_b, val_b, key_a, val_a)
            one_pass(22, 10, key_a, val_a, key_b, val_b,
                     exp_vals=True, last=True)

            @plsc.parallel_loop(0, NV)
            def _sufA(i):
                e = val_b[pl.ds(i * L, L)]
                out_v[pl.ds(i * L, L)] = lax.rev(
                    plsc.cumsum(lax.rev(e, (0,))), (0,))

            base_idx = lax.iota(jnp.int32, L) * L
            def sufB(mm, carry):
                m = NV // L - 1 - mm
                tot = plsc.load_gather(out_v, [m * (L * L) + base_idx])
                sfx = lax.rev(plsc.cumsum(lax.rev(tot, (0,))), (0,))
                car_v[pl.ds(m * L, L)] = sfx - tot + carry
                return carry + jnp.max(sfx)

            lax.fori_loop(0, NV // L, sufB, jnp.float32(0.0))

            @plsc.parallel_loop(0, NV)
            def _sufC(i):
                cv = car_v[pl.ds(i, L)]
                out_v[pl.ds(i * L, L)] = out_v[pl.ds(i * L, L)] + cv[0]

            pltpu.sync_copy(out_v, z_hbm.at[task])

    return k(predsT, targetsT)


def _tc_finish(z, preds):
    def body(z_ref, p_ref, o_ref):
        lz = jnp.log(z_ref[...] + jnp.float32(EPS))
        o_ref[0, 0] = (jnp.sum(lz) - jnp.sum(p_ref[...])) / jnp.float32(N * T)

    out = pl.pallas_call(
        body,
        out_shape=jax.ShapeDtypeStruct((1, 1), jnp.float32),
        out_specs=pl.BlockSpec(memory_space=pltpu.SMEM),
    )(z, preds)
    return out[0, 0]


def kernel(preds, targets):
    predsT = preds.T
    targetsT = targets.T
    z = _sc_zvalues(predsT, targetsT)
    return _tc_finish(z, preds)


# speedup vs baseline: 1.6828x; 1.1161x over previous
import functools

import jax
import jax.numpy as jnp
from jax import lax
from jax.experimental import pallas as pl
from jax.experimental.pallas import tpu as pltpu
from jax.experimental.pallas import tpu_sc as plsc

N = 16384
T = 16
L = 16
NV = N // L
R = 2048
EPS = 1e-12


def _sc_zvalues(predsT, targetsT):
    mesh = plsc.VectorSubcoreMesh(core_axis_name="c", subcore_axis_name="s")

    @functools.partial(
        pl.kernel,
        out_type=jax.ShapeDtypeStruct((T, N), jnp.float32),
        mesh=mesh,
        compiler_params=pltpu.CompilerParams(needs_layout_passes=False),
        scratch_types=[
            pltpu.VMEM((N,), jnp.float32),
            pltpu.VMEM((N,), jnp.float32),
            pltpu.VMEM((N,), jnp.int32),
            pltpu.VMEM((N,), jnp.int32),
            pltpu.VMEM((N,), jnp.float32),
            pltpu.VMEM((R,), jnp.int32),
            pltpu.VMEM((N,), jnp.float32),
            pltpu.VMEM((NV + L,), jnp.float32),
            pltpu.VMEM((N,), jnp.int32),
        ],
    )
    def k(predsT_hbm, targetsT_hbm, z_hbm, tgt_v, val_a, key_a, key_b, val_b,
          hist, out_v, car_v, aux_v):
        c = lax.axis_index("c")
        s = lax.axis_index("s")

        @pl.when(s < 8)
        def _():
            task = s * 2 + c
            pltpu.sync_copy(targetsT_hbm.at[task], tgt_v)
            pltpu.sync_copy(predsT_hbm.at[task], val_a)

            occ0, _ = plsc.scan_count(jnp.zeros((L,), jnp.int32))
            base0 = jnp.min(occ0)

            @plsc.parallel_loop(0, R // L)
            def _zh(j):
                hist[pl.ds(j * L, L)] = jnp.zeros((L,), jnp.int32)

            @plsc.parallel_loop(0, NV,
                                carry=jnp.full((L,), -jnp.inf, jnp.float32))
            def mx(i, acc):
                tv = tgt_v[pl.ds(i * L, L)]
                u = plsc.bitcast(tv, jnp.uint32)
                neg = (u >> 31) != 0
                key = jnp.where(neg, u, u ^ jnp.uint32(0x7FFFFFFF))
                key_a[pl.ds(i * L, L)] = plsc.bitcast(key, jnp.int32)
                d = (key & jnp.uint32(0x7FF)).astype(jnp.int32)
                occ, lastm = plsc.scan_count(d)
                occ = occ - base0
                aux_v[pl.ds(i * L, L)] = occ + jnp.where(lastm, 256, 0)
                plsc.addupdate_scatter(hist, [d], occ + 1, mask=lastm)
                return jnp.maximum(acc, val_a[pl.ds(i * L, L)])

            smax = jnp.max(mx)

            def one_pass(shift, nbits, src_k, src_v, dst_k, dst_v,
                         skip_hist=False, exp_vals=False, last=False):
                dmask = jnp.uint32((1 << nbits) - 1)

                if not skip_hist:
                    @plsc.parallel_loop(0, R // L)
                    def _zh2(j):
                        hist[pl.ds(j * L, L)] = jnp.zeros((L,), jnp.int32)

                    @plsc.parallel_loop(0, NV)
                    def _hb(i):
                        kk = plsc.bitcast(src_k[pl.ds(i * L, L)], jnp.uint32)
                        d = ((kk >> jnp.uint32(shift)) & dmask).astype(
                            jnp.int32)
                        occ, lastm = plsc.scan_count(d)
                        occ = occ - base0
                        aux_v[pl.ds(i * L, L)] = occ + jnp.where(lastm, 256, 0)
                        plsc.addupdate_scatter(hist, [d], occ + 1, mask=lastm)

                def sb(j, carry):
                    v = hist[pl.ds(j * L, L)]
                    cs = plsc.cumsum(v)
                    hist[pl.ds(j * L, L)] = cs - v + carry
                    return carry + jnp.sum(v)

                lax.fori_loop(0, R // L, sb, jnp.int32(0), unroll=4)

                def pf(i):
                    kk = src_k[pl.ds(i * L, L)]
                    vv = src_v[pl.ds(i * L, L)]
                    if exp_vals:
                        vv = jnp.exp(vv - smax)
                    ku = plsc.bitcast(kk, jnp.uint32)
                    d = ((ku >> jnp.uint32(shift)) & dmask).astype(jnp.int32)
                    aux = aux_v[pl.ds(i * L, L)]
                    occ = aux & 255
                    lastm = aux > 255
                    return kk, vv, d, occ, lastm

                def pb(i, st):
                    kk, vv, d, occ, lastm = st
                    base = plsc.load_gather(hist, [d])
                    pos = base + occ
                    if not last:
                        plsc.store_scatter(dst_k, [pos], kk)
                    plsc.store_scatter(dst_v, [pos], vv)
                    plsc.addupdate_scatter(hist, [d], occ + 1, mask=lastm)
                    return pf(jnp.minimum(i + 1, NV - 1))

                lax.fori_loop(0, NV, pb, pf(jnp.int32(0)), unroll=4)

            one_pass(0, 11, key_a, val_a, key_b, val_b, skip_hist=True)
            one_pass(11, 11, key_b, val_b, key_a, val_a)
            one_pass(22, 10, key_a, val_a, key_b, val_b,
                     exp_vals=True, last=True)

            @plsc.parallel_loop(0, NV)
            def _sufA(i):
                e = val_b[pl.ds(i * L, L)]
                out_v[pl.ds(i * L, L)] = lax.rev(
                    plsc.cumsum(lax.rev(e, (0,))), (0,))

            base_idx = lax.iota(jnp.int32, L) * L
            def sufB(mm, carry):
                m = NV // L - 1 - mm
                tot = plsc.load_gather(out_v, [m * (L * L) + base_idx])
                sfx = lax.rev(plsc.cumsum(lax.rev(tot, (0,))), (0,))
                car_v[pl.ds(m * L, L)] = sfx - tot + carry
                return carry + jnp.max(sfx)

            lax.fori_loop(0, NV // L, sufB, jnp.float32(0.0))

            @plsc.parallel_loop(0, NV)
            def _sufC(i):
                cv = car_v[pl.ds(i, L)]
                out_v[pl.ds(i * L, L)] = out_v[pl.ds(i * L, L)] + cv[0]

            pltpu.sync_copy(out_v, z_hbm.at[task])

    return k(predsT, targetsT)


def _tc_finish(z, preds):
    def body(z_ref, p_ref, o_ref):
        lz = jnp.log(z_ref[...] + jnp.float32(EPS))
        o_ref[0, 0] = (jnp.sum(lz) - jnp.sum(p_ref[...])) / jnp.float32(N * T)

    out = pl.pallas_call(
        body,
        out_shape=jax.ShapeDtypeStruct((1, 1), jnp.float32),
        out_specs=pl.BlockSpec(memory_space=pltpu.SMEM),
    )(z, preds)
    return out[0, 0]


def kernel(preds, targets):
    predsT = preds.T
    targetsT = targets.T
    z = _sc_zvalues(predsT, targetsT)
    return _tc_finish(z, preds)


# speedup vs baseline: 1.9480x; 1.1576x over previous
import functools

import jax
import jax.numpy as jnp
from jax import lax
from jax.experimental import pallas as pl
from jax.experimental.pallas import tpu as pltpu
from jax.experimental.pallas import tpu_sc as plsc

N = 16384
T = 16
L = 16
HN = N // 2
HNV = HN // L
PR = HN // 128
R = 2048
EPS = 1e-12


def _sc_zvalues(predsF, targetsF):
    mesh = plsc.VectorSubcoreMesh(core_axis_name="c", subcore_axis_name="s")

    @functools.partial(
        pl.kernel,
        out_type=jax.ShapeDtypeStruct((T * N,), jnp.float32),
        mesh=mesh,
        compiler_params=pltpu.CompilerParams(needs_layout_passes=False),
        scratch_types=[
            pltpu.VMEM((HN,), jnp.float32),
            pltpu.VMEM((HN,), jnp.float32),
            pltpu.VMEM((HN,), jnp.int32),
            pltpu.VMEM((HN,), jnp.int32),
            pltpu.VMEM((HN,), jnp.float32),
            pltpu.VMEM((HN,), jnp.float32),
            pltpu.VMEM((HN,), jnp.int32),
            pltpu.VMEM((PR, 128), jnp.int32),
            pltpu.VMEM((R,), jnp.int32),
            pltpu.VMEM((R,), jnp.int32),
            pltpu.VMEM((HN,), jnp.float32),
            pltpu.VMEM((HNV + L,), jnp.float32),
            pltpu.VMEM((L,), jnp.float32),
            pltpu.SemaphoreType.DMA,
            pltpu.VMEM_SHARED((8 * N,), jnp.int32),
            pltpu.VMEM_SHARED((8 * N,), jnp.float32),
            pltpu.VMEM_SHARED((16 * R,), jnp.int32),
            pltpu.VMEM_SHARED((16 * L,), jnp.float32),
        ],
    )
    def k(predsF_hbm, targetsF_hbm, z_hbm, tgt_v, val_a, key_a, key_b, val_b,
          e_v, aux_v, pos_v, hist, hist2, out_v, car_v, xch, sem,
          keyS, valS, histS, totS):
        c = lax.axis_index("c")
        s = lax.axis_index("s")
        tl = s % 8
        h = s // 8
        task = c * 8 + tl
        base_elem = task * N + h * HN
        sbase = tl * N
        slot = tl * 2 + h
        oslot = tl * 2 + (1 - h)

        pltpu.sync_copy(targetsF_hbm.at[pl.ds(base_elem, HN)], tgt_v)
        pltpu.sync_copy(predsF_hbm.at[pl.ds(base_elem, HN)], val_a)

        occ0, _ = plsc.scan_count(jnp.zeros((L,), jnp.int32))
        base0 = jnp.min(occ0)

        @plsc.parallel_loop(0, R // L)
        def _zh(j):
            hist[pl.ds(j * L, L)] = jnp.zeros((L,), jnp.int32)

        @plsc.parallel_loop(0, HNV,
                            carry=jnp.full((L,), -jnp.inf, jnp.float32))
        def mx(i, acc):
            tv = tgt_v[pl.ds(i * L, L)]
            u = plsc.bitcast(tv, jnp.uint32)
            neg = (u >> 31) != 0
            key = jnp.where(neg, u, u ^ jnp.uint32(0x7FFFFFFF))
            key_a[pl.ds(i * L, L)] = plsc.bitcast(key, jnp.int32)
            d = (key & jnp.uint32(0x7FF)).astype(jnp.int32)
            occ, lastm = plsc.scan_count(d)
            occ = occ - base0
            aux_v[pl.ds(i * L, L)] = occ + jnp.where(lastm, 256, 0)
            plsc.addupdate_scatter(hist, [d], occ + 1, mask=lastm)
            return jnp.maximum(acc, val_a[pl.ds(i * L, L)])

        pltpu.sync_copy(hist, histS.at[pl.ds(slot * R, R)])
        xch[...] = mx
        pltpu.sync_copy(xch, totS.at[pl.ds(slot * L, L)])
        plsc.subcore_barrier()
        pltpu.sync_copy(totS.at[pl.ds(oslot * L, L)], xch)
        smax = jnp.maximum(jnp.max(mx), jnp.max(xch[...]))

        def one_pass(shift, nbits, src_k, src_v, dst_k, dst_v,
                     skip_hist=False, last=False):
            dmask = jnp.uint32((1 << nbits) - 1)

            if not skip_hist:
                @plsc.parallel_loop(0, R // L)
                def _zh2(j):
                    hist[pl.ds(j * L, L)] = jnp.zeros((L,), jnp.int32)

                @plsc.parallel_loop(0, HNV)
                def _hb(i):
                    kk = plsc.bitcast(src_k[pl.ds(i * L, L)], jnp.uint32)
                    d = ((kk >> jnp.uint32(shift)) & dmask).astype(jnp.int32)
                    occ, lastm = plsc.scan_count(d)
                    occ = occ - base0
                    aux_v[pl.ds(i * L, L)] = occ + jnp.where(lastm, 256, 0)
                    plsc.addupdate_scatter(hist, [d], occ + 1, mask=lastm)

                pltpu.sync_copy(hist, histS.at[pl.ds(slot * R, R)])
                plsc.subcore_barrier()

            pltpu.sync_copy(histS.at[pl.ds(oslot * R, R)], hist2)

            def sb(j, carry):
                own = hist[pl.ds(j * L, L)]
                other = hist2[pl.ds(j * L, L)]
                tot = own + other
                cs = plsc.cumsum(tot)
                hist[pl.ds(j * L, L)] = cs - tot + carry + other * h
                return carry + jnp.sum(tot)

            lax.fori_loop(0, R // L, sb, sbase, unroll=4)

            if last:
                @plsc.parallel_loop(0, HNV)
                def _eb(i):
                    e_v[pl.ds(i * L, L)] = jnp.exp(
                        src_v[pl.ds(i * L, L)] - smax)

            def pb(i, _):
                kk = src_k[pl.ds(i * L, L)]
                ku = plsc.bitcast(kk, jnp.uint32)
                d = ((ku >> jnp.uint32(shift)) & dmask).astype(jnp.int32)
                aux = aux_v[pl.ds(i * L, L)]
                occ = aux & 255
                lastm = aux > 255
                base = plsc.load_gather(hist, [d])
                pos_v[i >> 3, pl.ds((i & 7) * L, L)] = base + occ
                plsc.addupdate_scatter(hist, [d], occ + 1, mask=lastm)
                return 0

            lax.fori_loop(0, HNV, pb, 0, unroll=4)

            vsrc = e_v if last else src_v

            def fire(j, _):
                if not last:
                    pltpu.async_copy(src_k.at[pl.ds(j * 128, 128)],
                                     keyS.at[pos_v.at[j]], sem)
                pltpu.async_copy(vsrc.at[pl.ds(j * 128, 128)],
                                 valS.at[pos_v.at[j]], sem)
                return 0

            lax.fori_loop(0, PR, fire, 0)

            def drain(j, _):
                if not last:
                    pltpu.make_async_copy(src_k.at[pl.ds(j * 128, 128)],
                                          keyS.at[pos_v.at[j]], sem).wait()
                pltpu.make_async_copy(vsrc.at[pl.ds(j * 128, 128)],
                                      valS.at[pos_v.at[j]], sem).wait()
                return 0

            lax.fori_loop(0, PR, drain, 0)
            plsc.subcore_barrier()

            half_base = sbase + h * HN
            if not last:
                pltpu.sync_copy(keyS.at[pl.ds(half_base, HN)], dst_k)
            pltpu.sync_copy(valS.at[pl.ds(half_base, HN)], dst_v)

        one_pass(0, 11, key_a, val_a, key_b, val_b, skip_hist=True)
        one_pass(11, 11, key_b, val_b, key_a, val_a)
        one_pass(22, 10, key_a, val_a, key_b, val_b, last=True)

        @plsc.parallel_loop(0, HNV)
        def _sufA(i):
            e = val_b[pl.ds(i * L, L)]
            out_v[pl.ds(i * L, L)] = lax.rev(
                plsc.cumsum(lax.rev(e, (0,))), (0,))

        base_idx = lax.iota(jnp.int32, L) * L

        def sufB(mm, carry):
            m = HNV // L - 1 - mm
            tot = plsc.load_gather(out_v, [m * (L * L) + base_idx])
            sfx = lax.rev(plsc.cumsum(lax.rev(tot, (0,))), (0,))
            car_v[pl.ds(m * L, L)] = sfx - tot + carry
            return carry + jnp.max(sfx)

        etot = lax.fori_loop(0, HNV // L, sufB, jnp.float32(0.0))

        xch[...] = jnp.full((L,), etot, jnp.float32)
        pltpu.sync_copy(xch, totS.at[pl.ds(slot * L, L)])
        plsc.subcore_barrier()
        pltpu.sync_copy(totS.at[pl.ds(oslot * L, L)], xch)
        xv = xch[...]
        extra = xv[0] * (1 - h).astype(jnp.float32)

        @plsc.parallel_loop(0, HNV)
        def _sufC(i):
            cv = car_v[pl.ds(i, L)]
            out_v[pl.ds(i * L, L)] = out_v[pl.ds(i * L, L)] + (cv[0] + extra)

        pltpu.sync_copy(out_v, z_hbm.at[pl.ds(base_elem, HN)])

    return k(predsF, targetsF)


def _tc_finish(z, preds):
    def body(z_ref, p_ref, o_ref):
        lz = jnp.log(z_ref[...] + jnp.float32(EPS))
        o_ref[0, 0] = (jnp.sum(lz) - jnp.sum(p_ref[...])) / jnp.float32(N * T)

    out = pl.pallas_call(
        body,
        out_shape=jax.ShapeDtypeStruct((1, 1), jnp.float32),
        out_specs=pl.BlockSpec(memory_space=pltpu.SMEM),
    )(z, preds)
    return out[0, 0]


def kernel(preds, targets):
    predsF = preds.T.reshape(T * N)
    targetsF = targets.T.reshape(T * N)
    z = _sc_zvalues(predsF, targetsF).reshape(T, N)
    return _tc_finish(z, preds)


# speedup vs baseline: 1.9504x; 1.0013x over previous
import functools

import jax
import jax.numpy as jnp
from jax import lax
from jax.experimental import pallas as pl
from jax.experimental.pallas import tpu as pltpu
from jax.experimental.pallas import tpu_sc as plsc

N = 16384
T = 16
L = 16
HN = N // 2
HNV = HN // L
PR = HN // 128
R = 2048
EPS = 1e-12


def _sc_zvalues(predsF, targetsF):
    mesh = plsc.VectorSubcoreMesh(core_axis_name="c", subcore_axis_name="s")

    @functools.partial(
        pl.kernel,
        out_type=jax.ShapeDtypeStruct((T * N,), jnp.float32),
        mesh=mesh,
        compiler_params=pltpu.CompilerParams(needs_layout_passes=False),
        scratch_types=[
            pltpu.VMEM((HN,), jnp.float32),
            pltpu.VMEM((HN,), jnp.float32),
            pltpu.VMEM((HN,), jnp.int32),
            pltpu.VMEM((HN,), jnp.int32),
            pltpu.VMEM((HN,), jnp.float32),
            pltpu.VMEM((HN,), jnp.float32),
            pltpu.VMEM((HN,), jnp.int32),
            pltpu.VMEM((PR, 128), jnp.int32),
            pltpu.VMEM((R,), jnp.int32),
            pltpu.VMEM((R,), jnp.int32),
            pltpu.VMEM((HN,), jnp.float32),
            pltpu.VMEM((HNV + L,), jnp.float32),
            pltpu.VMEM((L,), jnp.float32),
            pltpu.SemaphoreType.DMA,
            pltpu.VMEM_SHARED((8 * N,), jnp.int32),
            pltpu.VMEM_SHARED((8 * N,), jnp.float32),
            pltpu.VMEM_SHARED((16 * R,), jnp.int32),
            pltpu.VMEM_SHARED((16 * L,), jnp.float32),
        ],
    )
    def k(predsF_hbm, targetsF_hbm, z_hbm, tgt_v, val_a, key_a, key_b, val_b,
          e_v, aux_v, pos_v, hist, hist2, out_v, car_v, xch, sem,
          keyS, valS, histS, totS):
        c = lax.axis_index("c")
        s = lax.axis_index("s")
        tl = s % 8
        h = s // 8
        task = c * 8 + tl
        base_elem = task * N + h * HN
        sbase = tl * N
        slot = tl * 2 + h
        oslot = tl * 2 + (1 - h)

        pltpu.sync_copy(targetsF_hbm.at[pl.ds(base_elem, HN)], tgt_v)
        pltpu.sync_copy(predsF_hbm.at[pl.ds(base_elem, HN)], val_a)

        occ0, _ = plsc.scan_count(jnp.zeros((L,), jnp.int32))
        base0 = jnp.min(occ0)

        @plsc.parallel_loop(0, R // L)
        def _zh(j):
            hist[pl.ds(j * L, L)] = jnp.zeros((L,), jnp.int32)

        @plsc.parallel_loop(0, HNV,
                            carry=jnp.full((L,), -jnp.inf, jnp.float32))
        def mx(i, acc):
            tv = tgt_v[pl.ds(i * L, L)]
            u = plsc.bitcast(tv, jnp.uint32)
            neg = (u >> 31) != 0
            key = jnp.where(neg, u, u ^ jnp.uint32(0x7FFFFFFF))
            key_a[pl.ds(i * L, L)] = plsc.bitcast(key, jnp.int32)
            d = (key & jnp.uint32(0x7FF)).astype(jnp.int32)
            occ, lastm = plsc.scan_count(d)
            occ = occ - base0
            aux_v[pl.ds(i * L, L)] = (d + (occ << 11)
                                      + jnp.where(lastm, 1 << 15, 0))
            plsc.addupdate_scatter(hist, [d], occ + 1, mask=lastm)
            return jnp.maximum(acc, val_a[pl.ds(i * L, L)])

        pltpu.sync_copy(hist, histS.at[pl.ds(slot * R, R)])
        xch[...] = mx
        pltpu.sync_copy(xch, totS.at[pl.ds(slot * L, L)])
        plsc.subcore_barrier()
        pltpu.sync_copy(totS.at[pl.ds(oslot * L, L)], xch)
        smax = jnp.maximum(jnp.max(mx), jnp.max(xch[...]))

        def one_pass(shift, nbits, src_k, src_v, dst_k, dst_v,
                     skip_hist=False, last=False):
            dmask = jnp.uint32((1 << nbits) - 1)

            if not skip_hist:
                @plsc.parallel_loop(0, R // L)
                def _zh2(j):
                    hist[pl.ds(j * L, L)] = jnp.zeros((L,), jnp.int32)

                @plsc.parallel_loop(0, HNV)
                def _hb(i):
                    kk = plsc.bitcast(src_k[pl.ds(i * L, L)], jnp.uint32)
                    d = ((kk >> jnp.uint32(shift)) & dmask).astype(jnp.int32)
                    occ, lastm = plsc.scan_count(d)
                    occ = occ - base0
                    aux_v[pl.ds(i * L, L)] = (d + (occ << 11)
                                              + jnp.where(lastm, 1 << 15, 0))
                    plsc.addupdate_scatter(hist, [d], occ + 1, mask=lastm)

                pltpu.sync_copy(hist, histS.at[pl.ds(slot * R, R)])
                plsc.subcore_barrier()

            pltpu.sync_copy(histS.at[pl.ds(oslot * R, R)], hist2)

            def sb(j, carry):
                own = hist[pl.ds(j * L, L)]
                other = hist2[pl.ds(j * L, L)]
                tot = own + other
                cs = plsc.cumsum(tot)
                hist[pl.ds(j * L, L)] = cs - tot + carry + other * h
                return carry + jnp.sum(tot)

            lax.fori_loop(0, R // L, sb, sbase, unroll=4)

            if last:
                @plsc.parallel_loop(0, HNV)
                def _eb(i):
                    e_v[pl.ds(i * L, L)] = jnp.exp(
                        src_v[pl.ds(i * L, L)] - smax)

            def pb(i, _):
                aux = aux_v[pl.ds(i * L, L)]
                d = aux & 2047
                occ = (aux >> 11) & 15
                lastm = aux > 32767
                base = plsc.load_gather(hist, [d])
                pos_v[i >> 3, pl.ds((i & 7) * L, L)] = base + occ
                plsc.addupdate_scatter(hist, [d], occ + 1, mask=lastm)
                return 0

            lax.fori_loop(0, HNV, pb, 0, unroll=4)

            vsrc = e_v if last else src_v

            def fire(j, _):
                if not last:
                    pltpu.async_copy(src_k.at[pl.ds(j * 128, 128)],
                                     keyS.at[pos_v.at[j]], sem)
                pltpu.async_copy(vsrc.at[pl.ds(j * 128, 128)],
                                 valS.at[pos_v.at[j]], sem)
                return 0

            lax.fori_loop(0, PR, fire, 0)

            def drain(j, _):
                if not last:
                    pltpu.make_async_copy(src_k.at[pl.ds(j * 128, 128)],
                                          keyS.at[pos_v.at[j]], sem).wait()
                pltpu.make_async_copy(vsrc.at[pl.ds(j * 128, 128)],
                                      valS.at[pos_v.at[j]], sem).wait()
                return 0

            lax.fori_loop(0, PR, drain, 0)
            plsc.subcore_barrier()

            half_base = sbase + h * HN
            if not last:
                pltpu.sync_copy(keyS.at[pl.ds(half_base, HN)], dst_k)
            pltpu.sync_copy(valS.at[pl.ds(half_base, HN)], dst_v)

        one_pass(0, 11, key_a, val_a, key_b, val_b, skip_hist=True)
        one_pass(11, 11, key_b, val_b, key_a, val_a)
        one_pass(22, 10, key_a, val_a, key_b, val_b, last=True)

        @plsc.parallel_loop(0, HNV)
        def _sufA(i):
            e = val_b[pl.ds(i * L, L)]
            out_v[pl.ds(i * L, L)] = lax.rev(
                plsc.cumsum(lax.rev(e, (0,))), (0,))

        base_idx = lax.iota(jnp.int32, L) * L

        def sufB(mm, carry):
            m = HNV // L - 1 - mm
            tot = plsc.load_gather(out_v, [m * (L * L) + base_idx])
            sfx = lax.rev(plsc.cumsum(lax.rev(tot, (0,))), (0,))
            car_v[pl.ds(m * L, L)] = sfx - tot + carry
            return carry + jnp.max(sfx)

        etot = lax.fori_loop(0, HNV // L, sufB, jnp.float32(0.0))

        xch[...] = jnp.full((L,), etot, jnp.float32)
        pltpu.sync_copy(xch, totS.at[pl.ds(slot * L, L)])
        plsc.subcore_barrier()
        pltpu.sync_copy(totS.at[pl.ds(oslot * L, L)], xch)
        xv = xch[...]
        extra = xv[0] * (1 - h).astype(jnp.float32)

        @plsc.parallel_loop(0, HNV)
        def _sufC(i):
            cv = car_v[pl.ds(i, L)]
            out_v[pl.ds(i * L, L)] = out_v[pl.ds(i * L, L)] + (cv[0] + extra)

        pltpu.sync_copy(out_v, z_hbm.at[pl.ds(base_elem, HN)])

    return k(predsF, targetsF)


def _tc_finish(z, preds):
    def body(z_ref, p_ref, o_ref):
        lz = jnp.log(z_ref[...] + jnp.float32(EPS))
        o_ref[0, 0] = (jnp.sum(lz) - jnp.sum(p_ref[...])) / jnp.float32(N * T)

    out = pl.pallas_call(
        body,
        out_shape=jax.ShapeDtypeStruct((1, 1), jnp.float32),
        out_specs=pl.BlockSpec(memory_space=pltpu.SMEM),
    )(z, preds)
    return out[0, 0]


def kernel(preds, targets):
    predsF = preds.T.reshape(T * N)
    targetsF = targets.T.reshape(T * N)
    z = _sc_zvalues(predsF, targetsF).reshape(T, N)
    return _tc_finish(z, preds)


# speedup vs baseline: 2.0031x; 1.0270x over previous
import functools

import jax
import jax.numpy as jnp
from jax import lax
from jax.experimental import pallas as pl
from jax.experimental.pallas import tpu as pltpu
from jax.experimental.pallas import tpu_sc as plsc

N = 16384
T = 16
L = 16
HN = N // 2
HNV = HN // L
PR = HN // 128
R = 2048
EPS = 1e-12


def _sc_zvalues(predsF, targetsF):
    mesh = plsc.VectorSubcoreMesh(core_axis_name="c", subcore_axis_name="s")

    @functools.partial(
        pl.kernel,
        out_type=jax.ShapeDtypeStruct((T * N,), jnp.float32),
        mesh=mesh,
        compiler_params=pltpu.CompilerParams(needs_layout_passes=False),
        scratch_types=[
            pltpu.VMEM((HN,), jnp.float32),
            pltpu.VMEM((HN,), jnp.float32),
            pltpu.VMEM((HN,), jnp.int32),
            pltpu.VMEM((HN,), jnp.int32),
            pltpu.VMEM((HN,), jnp.float32),
            pltpu.VMEM((HN,), jnp.float32),
            pltpu.VMEM((HN,), jnp.int32),
            pltpu.VMEM((PR, 128), jnp.int32),
            pltpu.VMEM((R,), jnp.int32),
            pltpu.VMEM((R,), jnp.int32),
            pltpu.VMEM((HN,), jnp.float32),
            pltpu.VMEM((HNV + L,), jnp.float32),
            pltpu.VMEM((L,), jnp.float32),
            pltpu.SemaphoreType.DMA,
            pltpu.VMEM_SHARED((8 * N,), jnp.int32),
            pltpu.VMEM_SHARED((8 * N,), jnp.float32),
            pltpu.VMEM_SHARED((16 * R,), jnp.int32),
            pltpu.VMEM_SHARED((16 * L,), jnp.float32),
        ],
    )
    def k(predsF_hbm, targetsF_hbm, z_hbm, tgt_v, val_a, key_a, key_b, val_b,
          e_v, aux_v, pos_v, hist, hist2, out_v, car_v, xch, sem,
          keyS, valS, histS, totS):
        c = lax.axis_index("c")
        s = lax.axis_index("s")
        tl = s % 8
        h = s // 8
        task = c * 8 + tl
        base_elem = task * N + h * HN
        sbase = tl * N
        slot = tl * 2 + h
        oslot = tl * 2 + (1 - h)

        pltpu.sync_copy(targetsF_hbm.at[pl.ds(base_elem, HN)], tgt_v)
        pltpu.sync_copy(predsF_hbm.at[pl.ds(base_elem, HN)], val_a)

        occ0, _ = plsc.scan_count(jnp.zeros((L,), jnp.int32))
        base0 = jnp.min(occ0)

        @plsc.parallel_loop(0, R // L)
        def _zh(j):
            hist[pl.ds(j * L, L)] = jnp.zeros((L,), jnp.int32)

        @plsc.parallel_loop(0, HNV,
                            carry=jnp.full((L,), -jnp.inf, jnp.float32))
        def mx(i, acc):
            tv = tgt_v[pl.ds(i * L, L)]
            u = plsc.bitcast(tv, jnp.uint32)
            neg = (u >> 31) != 0
            key = jnp.where(neg, u, u ^ jnp.uint32(0x7FFFFFFF))
            key_a[pl.ds(i * L, L)] = plsc.bitcast(key, jnp.int32)
            d = (key & jnp.uint32(0x7FF)).astype(jnp.int32)
            occ, lastm = plsc.scan_count(d)
            occ = occ - base0
            aux_v[pl.ds(i * L, L)] = (d + (occ << 11)
                                      + jnp.where(lastm, 1 << 15, 0))
            plsc.addupdate_scatter(hist, [d], occ + 1, mask=lastm)
            return jnp.maximum(acc, val_a[pl.ds(i * L, L)])

        pltpu.sync_copy(hist, histS.at[pl.ds(slot * R, R)])
        xch[...] = mx
        pltpu.sync_copy(xch, totS.at[pl.ds(slot * L, L)])
        plsc.subcore_barrier()
        pltpu.sync_copy(totS.at[pl.ds(oslot * L, L)], xch)
        smax = jnp.maximum(jnp.max(mx), jnp.max(xch[...]))

        def one_pass(shift, nbits, src_k, src_v, dst_k, dst_v,
                     skip_hist=False, last=False):
            dmask = jnp.uint32((1 << nbits) - 1)

            if not skip_hist:
                @plsc.parallel_loop(0, R // L)
                def _zh2(j):
                    hist[pl.ds(j * L, L)] = jnp.zeros((L,), jnp.int32)

                @plsc.parallel_loop(0, HNV)
                def _hb(i):
                    kk = plsc.bitcast(src_k[pl.ds(i * L, L)], jnp.uint32)
                    d = ((kk >> jnp.uint32(shift)) & dmask).astype(jnp.int32)
                    occ, lastm = plsc.scan_count(d)
                    occ = occ - base0
                    aux_v[pl.ds(i * L, L)] = (d + (occ << 11)
                                              + jnp.where(lastm, 1 << 15, 0))
                    plsc.addupdate_scatter(hist, [d], occ + 1, mask=lastm)

                pltpu.sync_copy(hist, histS.at[pl.ds(slot * R, R)])
                plsc.subcore_barrier()

            pltpu.sync_copy(histS.at[pl.ds(oslot * R, R)], hist2)

            def sb(j, carry):
                own = hist[pl.ds(j * L, L)]
                other = hist2[pl.ds(j * L, L)]
                tot = own + other
                cs = plsc.cumsum(tot)
                hist[pl.ds(j * L, L)] = cs - tot + carry + other * h
                return carry + jnp.sum(tot)

            lax.fori_loop(0, R // L, sb, sbase, unroll=4)

            if last:
                @plsc.parallel_loop(0, HNV)
                def _eb(i):
                    e_v[pl.ds(i * L, L)] = jnp.exp(
                        src_v[pl.ds(i * L, L)] - smax)

            def pb(i, _):
                aux = aux_v[pl.ds(i * L, L)]
                d = aux & 2047
                occ = (aux >> 11) & 15
                lastm = aux > 32767
                base = plsc.load_gather(hist, [d])
                pos_v[i >> 3, pl.ds((i & 7) * L, L)] = base + occ
                plsc.addupdate_scatter(hist, [d], occ + 1, mask=lastm)
                return 0

            lax.fori_loop(0, HNV, pb, 0, unroll=4)

            vsrc = e_v if last else src_v

            def fire(j, _):
                if not last:
                    pltpu.async_copy(src_k.at[pl.ds(j * 128, 128)],
                                     keyS.at[pos_v.at[j]], sem)
                pltpu.async_copy(vsrc.at[pl.ds(j * 128, 128)],
                                 valS.at[pos_v.at[j]], sem)
                return 0

            lax.fori_loop(0, PR, fire, 0)

            def drain(j, _):
                if not last:
                    pltpu.make_async_copy(src_k.at[pl.ds(j * 128, 128)],
                                          keyS.at[pos_v.at[j]], sem).wait()
                pltpu.make_async_copy(vsrc.at[pl.ds(j * 128, 128)],
                                      valS.at[pos_v.at[j]], sem).wait()
                return 0

            lax.fori_loop(0, PR, drain, 0)
            plsc.subcore_barrier()

            half_base = sbase + h * HN
            if not last:
                pltpu.sync_copy(keyS.at[pl.ds(half_base, HN)], dst_k)
            pltpu.sync_copy(valS.at[pl.ds(half_base, HN)], dst_v)

        one_pass(0, 11, key_a, val_a, key_b, val_b, skip_hist=True)
        one_pass(11, 11, key_b, val_b, key_a, val_a)
        one_pass(22, 10, key_a, val_a, key_b, val_b, last=True)

        @plsc.parallel_loop(0, HNV)
        def _sufA(i):
            e = val_b[pl.ds(i * L, L)]
            out_v[pl.ds(i * L, L)] = lax.rev(
                plsc.cumsum(lax.rev(e, (0,))), (0,))

        base_idx = lax.iota(jnp.int32, L) * L

        def sufB(mm, carry):
            m = HNV // L - 1 - mm
            tot = plsc.load_gather(out_v, [m * (L * L) + base_idx])
            sfx = lax.rev(plsc.cumsum(lax.rev(tot, (0,))), (0,))
            car_v[pl.ds(m * L, L)] = sfx - tot + carry
            return carry + jnp.max(sfx)

        etot = lax.fori_loop(0, HNV // L, sufB, jnp.float32(0.0))

        xch[...] = jnp.full((L,), etot, jnp.float32)
        pltpu.sync_copy(xch, totS.at[pl.ds(slot * L, L)])
        plsc.subcore_barrier()
        pltpu.sync_copy(totS.at[pl.ds(oslot * L, L)], xch)
        xv = xch[...]
        extra = xv[0] * (1 - h).astype(jnp.float32)

        @plsc.parallel_loop(0, HNV)
        def _sufC(i):
            cv = car_v[pl.ds(i, L)]
            out_v[pl.ds(i * L, L)] = out_v[pl.ds(i * L, L)] + (cv[0] + extra)

        pltpu.sync_copy(out_v, z_hbm.at[pl.ds(base_elem, HN)])

    return k(predsF, targetsF)


def _tc_finish(z, preds):
    def body(z_ref, p_ref, o_ref):
        lz = jnp.log(z_ref[...] + jnp.float32(EPS))
        o_ref[0, 0] = (jnp.sum(lz) - jnp.sum(p_ref[...])) / jnp.float32(N * T)

    out = pl.pallas_call(
        body,
        out_shape=jax.ShapeDtypeStruct((1, 1), jnp.float32),
        out_specs=pl.BlockSpec(memory_space=pltpu.SMEM),
    )(z, preds)
    return out[0, 0]


def kernel(preds, targets):
    predsF = preds.T.reshape(T * N)
    targetsF = targets.T.reshape(T * N)
    z = _sc_zvalues(predsF, targetsF)
    return _tc_finish(z, preds)
